# baseline (speedup 1.0000x reference)
import functools

import jax
import jax.numpy as jnp
from jax import lax
from jax.experimental import pallas as pl
from jax.experimental.pallas import tpu as pltpu
from jax.experimental.pallas import tpu_sc as plsc

N = 10000
D = 128
E = 320000

NC = 2
NS = 16
NW = NC * NS

P = 10112
EPAD = 327680
CHUNK = 128
NCHUNK = EPAD // NW // CHUNK
NBUF = 4
SCH = 64
NSCH = EPAD // NW // SCH
RPT = P // NS

_mesh = plsc.VectorSubcoreMesh(core_axis_name="c", subcore_axis_name="s")


@functools.partial(
    pl.kernel,
    mesh=_mesh,
    out_type=[
        jax.ShapeDtypeStruct((P,), jnp.float32),
        jax.ShapeDtypeStruct((P,), jnp.float32),
    ],
    scratch_types=[
        pltpu.VMEM((NCHUNK, CHUNK), jnp.int32),
        pltpu.VMEM((CHUNK,), jnp.float32),
        pltpu.VMEM((CHUNK,), jnp.float32),
        pltpu.VMEM_SHARED((P,), jnp.float32),
        pltpu.SemaphoreType.DMA,
    ],
)
def _deg_kernel(dst_hbm, deg0_out, deg1_out, dstv, onesv, zv, deg_sp, dsem):
    c = lax.axis_index("c")
    s = lax.axis_index("s")
    w = s * NC + c
    pltpu.sync_copy(dst_hbm.at[w], dstv)

    def fill(i, carry):
        onesv[pl.ds(i * 16, 16)] = jnp.ones((16,), jnp.float32)
        zv[pl.ds(i * 16, 16)] = jnp.zeros((16,), jnp.float32)
        return carry

    lax.fori_loop(0, CHUNK // 16, fill, 0)

    nz = P // CHUNK
    k_lo = s * 5
    k_hi = jnp.minimum(nz, k_lo + 5)

    def zbody(k, carry):
        pltpu.sync_copy(zv, deg_sp.at[pl.ds(k * CHUNK, CHUNK)])
        return carry

    lax.fori_loop(k_lo, k_hi, zbody, 0)
    plsc.subcore_barrier()

    def body(j, carry):
        pltpu.async_copy(onesv, deg_sp.at[dstv.at[j]], dsem, add=True)
        return carry

    lax.fori_loop(0, NCHUNK, body, 0)

    def drain(j, carry):
        pltpu.make_async_copy(onesv, deg_sp.at[dstv.at[j]], dsem).wait()
        return carry

    lax.fori_loop(0, NCHUNK, drain, 0)
    plsc.subcore_barrier()

    def obody0(k, carry):
        pltpu.sync_copy(deg_sp.at[pl.ds(k * CHUNK, CHUNK)], deg0_out.at[pl.ds(k * CHUNK, CHUNK)])
        return carry

    def obody1(k, carry):
        pltpu.sync_copy(deg_sp.at[pl.ds(k * CHUNK, CHUNK)], deg1_out.at[pl.ds(k * CHUNK, CHUNK)])
        return carry

    @pl.when(c == 0)
    def _():
        lax.fori_loop(k_lo, k_hi, obody0, 0)

    @pl.when(c == 1)
    def _():
        lax.fori_loop(k_lo, k_hi, obody1, 0)


@functools.partial(
    pl.kernel,
    mesh=_mesh,
    out_type=jax.ShapeDtypeStruct((NC, P, D), jnp.float32),
    scratch_types=[
        pltpu.VMEM((NCHUNK, CHUNK), jnp.int32),
        pltpu.VMEM((NBUF, SCH), jnp.int32),
        pltpu.VMEM((NBUF, SCH), jnp.int32),
        pltpu.VMEM((NBUF, SCH, D), jnp.float32),
        pltpu.VMEM_SHARED((P, D), jnp.float32),
        [pltpu.SemaphoreType.DMA] * NBUF,
        [pltpu.SemaphoreType.DMA] * NBUF,
    ],
)
def _agg_kernel(hs_hbm, pk_hbm, zeros_hbm, acc_out, pkv, sidx, didx, rowbuf, acc_sp, gsem, ssem):
    c = lax.axis_index("c")
    s = lax.axis_index("s")
    w = s * NC + c
    rows0 = s * RPT
    pltpu.sync_copy(pk_hbm.at[w], pkv)
    pltpu.sync_copy(zeros_hbm, acc_sp.at[pl.ds(rows0, RPT)])

    def unpack(cj, half, b):
        row = cj // 2
        for i in range(SCH // 16):
            pk = pkv[row, pl.ds(half * SCH + i * 16, 16)]
            sidx[b, pl.ds(i * 16, 16)] = pk >> 16
            didx[b, pl.ds(i * 16, 16)] = pk & 0xFFFF

    plsc.subcore_barrier()

    for b in range(2):
        unpack(b, b % 2, b)
        pltpu.async_copy(hs_hbm.at[sidx.at[b]], rowbuf.at[b], gsem[b])

    def body(it, carry):
        for b in range(NBUF):
            cj = NBUF * it + b
            bn = (b + 2) % NBUF

            @pl.when(cj >= 2)
            def _():
                pltpu.make_async_copy(rowbuf.at[bn], acc_sp.at[didx.at[bn]], ssem[bn]).wait()

            @pl.when(cj + 2 < NSCH)
            def _():
                unpack(cj + 2, b % 2, bn)

            pltpu.make_async_copy(hs_hbm.at[sidx.at[b]], rowbuf.at[b], gsem[b]).wait()
            pltpu.async_copy(rowbuf.at[b], acc_sp.at[didx.at[b]], ssem[b], add=True)

            @pl.when(cj + 2 < NSCH)
            def _():
                pltpu.async_copy(hs_hbm.at[sidx.at[bn]], rowbuf.at[bn], gsem[bn])

        return carry

    lax.fori_loop(0, NSCH // NBUF, body, 0)
    for b in (2, 3):
        pltpu.make_async_copy(rowbuf.at[b], acc_sp.at[didx.at[b]], ssem[b]).wait()
    plsc.subcore_barrier()
    pltpu.sync_copy(acc_sp.at[pl.ds(rows0, RPT)], acc_out.at[c, pl.ds(rows0, RPT)])


BLK = 1264


def _row_mask(shape):
    i = pl.program_id(0)
    ridx = jax.lax.broadcasted_iota(jnp.int32, shape, 0) + i * BLK
    return ridx < N


def _pre_body(x_ref, w_ref, dp_ref, dis_ref, hw_ref, hs_ref):
    dp = dp_ref[...]
    deg = dp[:, 0] + dp[:, 1] + 1.0
    mask = _row_mask((BLK, 1))
    dis = jnp.where(mask, lax.rsqrt(deg)[:, None], 0.0)
    hw = jnp.dot(x_ref[...], w_ref[...], preferred_element_type=jnp.float32)
    hw_ref[...] = hw
    hs_ref[...] = jnp.where(mask, hw * dis, 0.0)
    dis_ref[...] = dis


def _mid_body(acc_ref, hw_ref, dis_ref, b_ref, wn_ref, hwn_ref, hsn_ref):
    agg = acc_ref[0] + acc_ref[1]
    dis = dis_ref[...]
    mask = _row_mask((BLK, 1))
    o = jnp.where(mask, jnp.maximum(dis * (agg + dis * hw_ref[...]) + b_ref[...], 0.0), 0.0)
    hwn = jnp.dot(o, wn_ref[...], preferred_element_type=jnp.float32)
    hwn_ref[...] = hwn
    hsn_ref[...] = jnp.where(mask, hwn * dis, 0.0)


def _post_body(acc_ref, hw_ref, dis_ref, b_ref, out_ref):
    agg = acc_ref[0] + acc_ref[1]
    dis = dis_ref[...]
    out_ref[...] = dis * (agg + dis * hw_ref[...]) + b_ref[...]


_pre = pl.pallas_call(
    _pre_body,
    grid=(P // BLK,),
    in_specs=[
        pl.BlockSpec((BLK, D), lambda i: (i, 0)),
        pl.BlockSpec((D, D), lambda i: (0, 0)),
        pl.BlockSpec((BLK, 2), lambda i: (i, 0)),
    ],
    name="gcn_pre",
    out_specs=[
        pl.BlockSpec((BLK, 1), lambda i: (i, 0)),
        pl.BlockSpec((BLK, D), lambda i: (i, 0)),
        pl.BlockSpec((BLK, D), lambda i: (i, 0)),
    ],
    out_shape=[
        jax.ShapeDtypeStruct((P, 1), jnp.float32),
        jax.ShapeDtypeStruct((P, D), jnp.float32),
        jax.ShapeDtypeStruct((P, D), jnp.float32),
    ],
)

_mid = pl.pallas_call(
    _mid_body,
    grid=(P // BLK,),
    in_specs=[
        pl.BlockSpec((2, BLK, D), lambda i: (0, i, 0)),
        pl.BlockSpec((BLK, D), lambda i: (i, 0)),
        pl.BlockSpec((BLK, 1), lambda i: (i, 0)),
        pl.BlockSpec((1, D), lambda i: (0, 0)),
        pl.BlockSpec((D, D), lambda i: (0, 0)),
    ],
    out_specs=[
        pl.BlockSpec((BLK, D), lambda i: (i, 0)),
        pl.BlockSpec((BLK, D), lambda i: (i, 0)),
    ],
    out_shape=[
        jax.ShapeDtypeStruct((P, D), jnp.float32),
        jax.ShapeDtypeStruct((P, D), jnp.float32),
    ],
)

_post = pl.pallas_call(
    _post_body,
    grid=(P // BLK,),
    in_specs=[
        pl.BlockSpec((2, BLK, D), lambda i: (0, i, 0)),
        pl.BlockSpec((BLK, D), lambda i: (i, 0)),
        pl.BlockSpec((BLK, 1), lambda i: (i, 0)),
        pl.BlockSpec((1, D), lambda i: (0, 0)),
    ],
    out_specs=pl.BlockSpec((BLK, D), lambda i: (i, 0)),
    out_shape=jax.ShapeDtypeStruct((N, D), jnp.float32),
)


def kernel(x, edge_index, W1, b1, W2, b2, W3, b3):
    xp = x
    src = edge_index[0]
    dst = edge_index[1]
    npad = EPAD - E
    padv = N + (jnp.arange(npad, dtype=jnp.int32) % (P - N))
    srcp = jnp.concatenate([src, padv])
    dstp_flat = jnp.concatenate([dst, padv])
    pk = ((srcp << 16) | dstp_flat).reshape(NW, NCHUNK, CHUNK)
    dstp = dstp_flat.reshape(NW, NCHUNK, CHUNK)
    zerosD = jnp.zeros((RPT, D), jnp.float32)

    deg0, deg1 = _deg_kernel(dstp)
    dis, hw, hs = _pre(xp, W1, jnp.stack([deg0, deg1], axis=-1))
    acc = _agg_kernel(hs, pk, zerosD)
    hw, hs = _mid(acc, hw, dis, b1.reshape(1, D), W2)
    acc = _agg_kernel(hs, pk, zerosD)
    hw, hs = _mid(acc, hw, dis, b2.reshape(1, D), W3)
    acc = _agg_kernel(hs, pk, zerosD)
    return _post(acc, hw, dis, b3.reshape(1, D))


# speedup vs baseline: 1.1189x; 1.1189x over previous
import functools

import jax
import jax.numpy as jnp
from jax import lax
from jax.experimental import pallas as pl
from jax.experimental.pallas import tpu as pltpu
from jax.experimental.pallas import tpu_sc as plsc

N = 10000
D = 128
E = 320000

NC = 2
NS = 16
NW = NC * NS

P = 10112
EPAD = 327680
CHUNK = 128
NCHUNK = EPAD // NW // CHUNK
NBUF = 2
RPT = P // NS

_mesh = plsc.VectorSubcoreMesh(core_axis_name="c", subcore_axis_name="s")


@functools.partial(
    pl.kernel,
    mesh=_mesh,
    out_type=[
        jax.ShapeDtypeStruct((P,), jnp.float32),
        jax.ShapeDtypeStruct((P,), jnp.float32),
    ],
    scratch_types=[
        pltpu.VMEM((NCHUNK, CHUNK), jnp.int32),
        pltpu.VMEM((CHUNK,), jnp.float32),
        pltpu.VMEM((CHUNK,), jnp.float32),
        pltpu.VMEM_SHARED((P,), jnp.float32),
        pltpu.SemaphoreType.DMA,
    ],
)
def _deg_kernel(dst_hbm, deg0_out, deg1_out, dstv, onesv, zv, deg_sp, dsem):
    c = lax.axis_index("c")
    s = lax.axis_index("s")
    w = s * NC + c
    pltpu.sync_copy(dst_hbm.at[w], dstv)

    def fill(i, carry):
        onesv[pl.ds(i * 16, 16)] = jnp.ones((16,), jnp.float32)
        zv[pl.ds(i * 16, 16)] = jnp.zeros((16,), jnp.float32)
        return carry

    lax.fori_loop(0, CHUNK // 16, fill, 0)

    nz = P // CHUNK
    k_lo = s * 5
    k_hi = jnp.minimum(nz, k_lo + 5)

    def zbody(k, carry):
        pltpu.sync_copy(zv, deg_sp.at[pl.ds(k * CHUNK, CHUNK)])
        return carry

    lax.fori_loop(k_lo, k_hi, zbody, 0)
    plsc.subcore_barrier()

    def body(j, carry):
        pltpu.async_copy(onesv, deg_sp.at[dstv.at[j]], dsem, add=True)
        return carry

    lax.fori_loop(0, NCHUNK, body, 0)

    def drain(j, carry):
        pltpu.make_async_copy(onesv, deg_sp.at[dstv.at[j]], dsem).wait()
        return carry

    lax.fori_loop(0, NCHUNK, drain, 0)
    plsc.subcore_barrier()

    def obody0(k, carry):
        pltpu.sync_copy(deg_sp.at[pl.ds(k * CHUNK, CHUNK)], deg0_out.at[pl.ds(k * CHUNK, CHUNK)])
        return carry

    def obody1(k, carry):
        pltpu.sync_copy(deg_sp.at[pl.ds(k * CHUNK, CHUNK)], deg1_out.at[pl.ds(k * CHUNK, CHUNK)])
        return carry

    @pl.when(c == 0)
    def _():
        lax.fori_loop(k_lo, k_hi, obody0, 0)

    @pl.when(c == 1)
    def _():
        lax.fori_loop(k_lo, k_hi, obody1, 0)


@functools.partial(
    pl.kernel,
    mesh=_mesh,
    out_type=jax.ShapeDtypeStruct((NC, P, D), jnp.float32),
    scratch_types=[
        pltpu.VMEM((NCHUNK, CHUNK), jnp.int32),
        pltpu.VMEM((NBUF, CHUNK), jnp.int32),
        pltpu.VMEM((NBUF, CHUNK), jnp.int32),
        pltpu.VMEM((NBUF, CHUNK, D), jnp.float32),
        pltpu.VMEM_SHARED((P, D), jnp.float32),
        [pltpu.SemaphoreType.DMA] * NBUF,
        [pltpu.SemaphoreType.DMA] * NBUF,
    ],
)
def _agg_kernel(hs_hbm, pk_hbm, zeros_hbm, acc_out, pkv, sidx, didx, rowbuf, acc_sp, gsem, ssem):
    c = lax.axis_index("c")
    s = lax.axis_index("s")
    w = s * NC + c
    rows0 = s * RPT
    pltpu.sync_copy(pk_hbm.at[w], pkv)
    pltpu.sync_copy(zeros_hbm, acc_sp.at[pl.ds(rows0, RPT)])

    def unpack(jj, b):
        for i in range(CHUNK // 16):
            pk = pkv[jj, pl.ds(i * 16, 16)]
            sidx[b, pl.ds(i * 16, 16)] = pk >> 16
            didx[b, pl.ds(i * 16, 16)] = pk & 0xFFFF

    plsc.subcore_barrier()

    for b in range(NBUF):
        unpack(b, b)
        pltpu.async_copy(hs_hbm.at[sidx.at[b]], rowbuf.at[b], gsem[b])

    def body(j4, carry):
        for b in range(NBUF):
            j = NBUF * j4 + b
            pltpu.make_async_copy(hs_hbm.at[sidx.at[b]], rowbuf.at[b], gsem[b]).wait()
            pltpu.async_copy(rowbuf.at[b], acc_sp.at[didx.at[b]], ssem[b], add=True)

            @pl.when(j4 < NCHUNK // NBUF - 1)
            def _():
                pltpu.make_async_copy(rowbuf.at[b], acc_sp.at[didx.at[b]], ssem[b]).wait()
                unpack(j + NBUF, b)
                pltpu.async_copy(hs_hbm.at[sidx.at[b]], rowbuf.at[b], gsem[b])

        return carry

    lax.fori_loop(0, NCHUNK // NBUF, body, 0)
    for b in range(NBUF):
        pltpu.make_async_copy(rowbuf.at[b], acc_sp.at[didx.at[b]], ssem[b]).wait()
    plsc.subcore_barrier()
    pltpu.sync_copy(acc_sp.at[pl.ds(rows0, RPT)], acc_out.at[c, pl.ds(rows0, RPT)])


BLK = 5056


def _row_mask(shape):
    i = pl.program_id(0)
    ridx = jax.lax.broadcasted_iota(jnp.int32, shape, 0) + i * BLK
    return ridx < N


def _pre_body(x_ref, w_ref, dp_ref, dis_ref, hw_ref, hs_ref):
    dp = dp_ref[...]
    deg = dp[:, 0] + dp[:, 1] + 1.0
    mask = _row_mask((BLK, 1))
    dis = jnp.where(mask, lax.rsqrt(deg)[:, None], 0.0)
    hw = jnp.dot(x_ref[...], w_ref[...], preferred_element_type=jnp.float32)
    hw_ref[...] = hw
    hs_ref[...] = jnp.where(mask, hw * dis, 0.0)
    dis_ref[...] = dis


def _mid_body(acc_ref, hw_ref, dis_ref, b_ref, wn_ref, hwn_ref, hsn_ref):
    agg = acc_ref[0] + acc_ref[1]
    dis = dis_ref[...]
    mask = _row_mask((BLK, 1))
    o = jnp.where(mask, jnp.maximum(dis * (agg + dis * hw_ref[...]) + b_ref[...], 0.0), 0.0)
    hwn = jnp.dot(o, wn_ref[...], preferred_element_type=jnp.float32)
    hwn_ref[...] = hwn
    hsn_ref[...] = jnp.where(mask, hwn * dis, 0.0)


def _post_body(acc_ref, hw_ref, dis_ref, b_ref, out_ref):
    agg = acc_ref[0] + acc_ref[1]
    dis = dis_ref[...]
    out_ref[...] = dis * (agg + dis * hw_ref[...]) + b_ref[...]


_pre = pl.pallas_call(
    _pre_body,
    grid=(P // BLK,),
    in_specs=[
        pl.BlockSpec((BLK, D), lambda i: (i, 0)),
        pl.BlockSpec((D, D), lambda i: (0, 0)),
        pl.BlockSpec((BLK, 2), lambda i: (i, 0)),
    ],
    name="gcn_pre",
    out_specs=[
        pl.BlockSpec((BLK, 1), lambda i: (i, 0)),
        pl.BlockSpec((BLK, D), lambda i: (i, 0)),
        pl.BlockSpec((BLK, D), lambda i: (i, 0)),
    ],
    out_shape=[
        jax.ShapeDtypeStruct((P, 1), jnp.float32),
        jax.ShapeDtypeStruct((P, D), jnp.float32),
        jax.ShapeDtypeStruct((P, D), jnp.float32),
    ],
)

_mid = pl.pallas_call(
    _mid_body,
    grid=(P // BLK,),
    in_specs=[
        pl.BlockSpec((2, BLK, D), lambda i: (0, i, 0)),
        pl.BlockSpec((BLK, D), lambda i: (i, 0)),
        pl.BlockSpec((BLK, 1), lambda i: (i, 0)),
        pl.BlockSpec((1, D), lambda i: (0, 0)),
        pl.BlockSpec((D, D), lambda i: (0, 0)),
    ],
    out_specs=[
        pl.BlockSpec((BLK, D), lambda i: (i, 0)),
        pl.BlockSpec((BLK, D), lambda i: (i, 0)),
    ],
    out_shape=[
        jax.ShapeDtypeStruct((P, D), jnp.float32),
        jax.ShapeDtypeStruct((P, D), jnp.float32),
    ],
)

_post = pl.pallas_call(
    _post_body,
    grid=(P // BLK,),
    in_specs=[
        pl.BlockSpec((2, BLK, D), lambda i: (0, i, 0)),
        pl.BlockSpec((BLK, D), lambda i: (i, 0)),
        pl.BlockSpec((BLK, 1), lambda i: (i, 0)),
        pl.BlockSpec((1, D), lambda i: (0, 0)),
    ],
    out_specs=pl.BlockSpec((BLK, D), lambda i: (i, 0)),
    out_shape=jax.ShapeDtypeStruct((N, D), jnp.float32),
)


def kernel(x, edge_index, W1, b1, W2, b2, W3, b3):
    xp = x
    src = edge_index[0]
    dst = edge_index[1]
    npad = EPAD - E
    padv = N + (jnp.arange(npad, dtype=jnp.int32) % (P - N))
    srcp = jnp.concatenate([src, padv])
    dstp_flat = jnp.concatenate([dst, padv])
    pk = ((srcp << 16) | dstp_flat).reshape(NW, NCHUNK, CHUNK)
    dstp = dstp_flat.reshape(NW, NCHUNK, CHUNK)
    zerosD = jnp.zeros((RPT, D), jnp.float32)

    deg0, deg1 = _deg_kernel(dstp)
    dis, hw, hs = _pre(xp, W1, jnp.stack([deg0, deg1], axis=-1))
    acc = _agg_kernel(hs, pk, zerosD)
    hw, hs = _mid(acc, hw, dis, b1.reshape(1, D), W2)
    acc = _agg_kernel(hs, pk, zerosD)
    hw, hs = _mid(acc, hw, dis, b2.reshape(1, D), W3)
    acc = _agg_kernel(hs, pk, zerosD)
    return _post(acc, hw, dis, b3.reshape(1, D))
